# baseline (speedup 1.0000x reference)
import functools

import jax
import jax.numpy as jnp
from jax import lax
from jax.experimental import pallas as pl
from jax.experimental.pallas import tpu as pltpu
from jax.experimental.pallas import tpu_sc as plsc

B_ = 4
M_ = 16384
N_ = 16384
NNZ_ = 262144
W_PRIMAL, W_DUAL, W_STAT, W_COMP = 0.1, 0.1, 0.6, 0.2

NC = 2
NS = 16
L = 16

SLICE = M_ // NS
ENT_PER_TILE = NNZ_ // NS
CHUNK = 2048
NCHUNK = ENT_PER_TILE // CHUNK


def _body(x_ref, lam_ref, rows_ref, cols_ref, vals_ref, b_ref, c_ref,
          out_ref,
          x_v, lam_v, ax_acc, atl_acc, rows_v, cols_v, vals_v,
          red_v, b_v, c_v, lamc_v, loss_v,
          stage_s):
    c = lax.axis_index("c")
    s = lax.axis_index("s")
    zf = jnp.zeros((L,), jnp.float32)

    for l in range(4):
        loss_v[l, :] = zf

    for bi in range(2):
        i = c * 2 + bi
        pltpu.sync_copy(x_ref.at[i], x_v)
        pltpu.sync_copy(lam_ref.at[i], lam_v)

        def zbody(k, carry):
            ax_acc[pl.ds(k * L, L)] = zf
            atl_acc[pl.ds(k * L, L)] = zf
            return carry

        lax.fori_loop(0, M_ // L, zbody, 0, unroll=8)

        for ch in range(NCHUNK):
            e0 = s * ENT_PER_TILE + ch * CHUNK
            pltpu.sync_copy(rows_ref.at[i, pl.ds(e0, CHUNK)], rows_v)
            pltpu.sync_copy(cols_ref.at[i, pl.ds(e0, CHUNK)], cols_v)
            pltpu.sync_copy(vals_ref.at[i, pl.ds(e0, CHUNK)], vals_v)

            def chunk_body(k, carry):
                cvec = cols_v[pl.ds(k * L, L)]
                rvec = rows_v[pl.ds(k * L, L)]
                vvec = vals_v[pl.ds(k * L, L)]
                xg = plsc.load_gather(x_v, [cvec])
                lg = plsc.load_gather(lam_v, [rvec])
                plsc.addupdate_scatter(ax_acc, [rvec], vvec * xg)
                plsc.addupdate_scatter(atl_acc, [cvec], vvec * lg)
                return carry

            lax.fori_loop(0, CHUNK // L, chunk_body, 0, unroll=4)

        pltpu.sync_copy(ax_acc, stage_s.at[s, pl.ds(0, M_)])
        pltpu.sync_copy(atl_acc, stage_s.at[s, pl.ds(M_, N_)])
        plsc.subcore_barrier()

        off = s * SLICE
        pltpu.sync_copy(b_ref.at[i, pl.ds(off, SLICE)], b_v)
        pltpu.sync_copy(lam_ref.at[i, pl.ds(off, SLICE)], lamc_v)
        pltpu.sync_copy(c_ref.at[i, pl.ds(off, SLICE)], c_v)

        for t in range(NS):
            pltpu.sync_copy(stage_s.at[t, pl.ds(off, SLICE)],
                            red_v.at[t])

        def ax_loss_body(k, accs):
            ap, ad, ac = accs
            tot = red_v[0, pl.ds(k * L, L)]
            for t in range(1, NS):
                tot = tot + red_v[t, pl.ds(k * L, L)]
            bb = b_v[pl.ds(k * L, L)]
            ll = lamc_v[pl.ds(k * L, L)]
            r = tot - bb
            p = jnp.maximum(r, 0.0)
            dn = jnp.maximum(-ll, 0.0)
            cm = ll * r
            return (ap + p * p, ad + dn * dn, ac + cm * cm)

        ap, ad, ac = lax.fori_loop(0, SLICE // L, ax_loss_body,
                                   (zf, zf, zf), unroll=2)

        for t in range(NS):
            pltpu.sync_copy(stage_s.at[t, pl.ds(M_ + off, SLICE)],
                            red_v.at[t])

        def atl_loss_body(k, ast):
            tot = red_v[0, pl.ds(k * L, L)]
            for t in range(1, NS):
                tot = tot + red_v[t, pl.ds(k * L, L)]
            cc = c_v[pl.ds(k * L, L)]
            st = tot + cc
            return ast + st * st

        ast = lax.fori_loop(0, SLICE // L, atl_loss_body, zf, unroll=2)

        loss_v[0, :] = loss_v[0, :] + ap
        loss_v[1, :] = loss_v[1, :] + ad
        loss_v[2, :] = loss_v[2, :] + ast
        loss_v[3, :] = loss_v[3, :] + ac
        plsc.subcore_barrier()

    pltpu.sync_copy(loss_v, out_ref.at[c, s])


_sc_kernel = functools.partial(
    pl.kernel,
    out_type=jax.ShapeDtypeStruct((NC, NS, 4, L), jnp.float32),
    mesh=plsc.VectorSubcoreMesh(core_axis_name="c", subcore_axis_name="s"),
    compiler_params=pltpu.CompilerParams(needs_layout_passes=False),
    scratch_types=[
        pltpu.VMEM((N_,), jnp.float32),
        pltpu.VMEM((M_,), jnp.float32),
        pltpu.VMEM((M_,), jnp.float32),
        pltpu.VMEM((N_,), jnp.float32),
        pltpu.VMEM((CHUNK,), jnp.int32),
        pltpu.VMEM((CHUNK,), jnp.int32),
        pltpu.VMEM((CHUNK,), jnp.float32),
        pltpu.VMEM((NS, SLICE), jnp.float32),
        pltpu.VMEM((SLICE,), jnp.float32),
        pltpu.VMEM((SLICE,), jnp.float32),
        pltpu.VMEM((SLICE,), jnp.float32),
        pltpu.VMEM((4, L), jnp.float32),
        pltpu.VMEM_SHARED((NS, M_ + N_), jnp.float32),
    ],
)(_body)


def kernel(x_hat, lam_hat, A_rows, A_cols, A_vals, b_pad, c_pad, b_mask, c_mask):
    x2 = x_hat.astype(jnp.float32).reshape(B_, N_)
    lam2 = lam_hat.astype(jnp.float32).reshape(B_, M_)
    part = _sc_kernel(x2, lam2, A_rows.astype(jnp.int32),
                      A_cols.astype(jnp.int32), A_vals.astype(jnp.float32),
                      b_pad.astype(jnp.float32), c_pad.astype(jnp.float32))
    sums = part.sum(axis=(0, 1, 3))
    total = (W_PRIMAL * sums[0] / M_ + W_DUAL * sums[1] / M_
             + W_STAT * sums[2] / N_ + W_COMP * sums[3] / M_) / B_
    return total.astype(jnp.float32)
